# baseline (speedup 1.0000x reference)
import jax
import jax.numpy as jnp
from jax.experimental import pallas as pl
from jax.experimental.pallas import tpu as pltpu

_N = 1024
_D = 16
_NG = _N // 8
_S = 256
_NT = _N // _S


def _dot(a, b):
    return jax.lax.dot_general(
        a, b, (((a.ndim - 1,), (0,)), ((), ())),
        preferred_element_type=jnp.float32,
        precision=jax.lax.Precision.HIGHEST)


def _edot(a, b):
    return jax.lax.dot_general(
        a, b, (((a.ndim - 1,), (0,)), ((), ())),
        preferred_element_type=jnp.float32)


def _gn_body(x_ref, u_ref,
             w1ee_ref, b1ee_ref, w2ee_ref, b2ee_ref,
             w1eu_ref, b1eu_ref, w2eu_ref, b2eu_ref,
             w1c0e_ref, b1c0e_ref, w2c0e_ref, b2c0e_ref,
             w1c0v_ref, b1c0v_ref, w2c0v_ref, b2c0v_ref,
             w1c0u_ref, b1c0u_ref, w2c0u_ref, b2c0u_ref,
             w1c1e_ref, b1c1e_ref, w2c1e_ref, b2c1e_ref,
             w1c1v_ref, b1c1v_ref, w2c1v_ref, b2c1v_ref,
             w1c1u_ref, b1c1u_ref, w2c1u_ref, b2c1u_ref,
             w1d_ref, b1d_ref, w2d_ref, b2d_ref,
             out_ref, a0t_ref, a1t_ref, a2t_ref):
    f32 = jnp.float32
    x = x_ref[...]
    u0 = u_ref[...]

    w1ee = w1ee_ref[...]
    b1ee = b1ee_ref[...]
    w2ee = w2ee_ref[...]
    b2ee = b2ee_ref[...]
    w1eu = w1eu_ref[...]
    b1eu = b1eu_ref[...]
    w2eu = w2eu_ref[...]
    b2eu = b2eu_ref[...]
    w1c0e = w1c0e_ref[...]
    b1c0e = b1c0e_ref[...]
    w2c0e = w2c0e_ref[...]
    b2c0e = b2c0e_ref[...]
    w1c0v = w1c0v_ref[...]
    b1c0v = b1c0v_ref[...]
    w2c0v = w2c0v_ref[...]
    b2c0v = b2c0v_ref[...]
    w1c0u = w1c0u_ref[...]
    b1c0u = b1c0u_ref[...]
    w2c0u = w2c0u_ref[...]
    b2c0u = b2c0u_ref[...]
    w1c1e = w1c1e_ref[...]
    b1c1e = b1c1e_ref[...]
    w2c1e = w2c1e_ref[...]
    b2c1e = b2c1e_ref[...]
    w1c1v = w1c1v_ref[...]
    b1c1v = b1c1v_ref[...]
    w2c1v = w2c1v_ref[...]
    b2c1v = b2c1v_ref[...]
    w1c1u = w1c1u_ref[...]
    b1c1u = b1c1u_ref[...]
    w2c1u = w2c1u_ref[...]
    b2c1u = b2c1u_ref[...]
    w1d = w1d_ref[...]
    b1d = b1d_ref[...]
    w2d = w2d_ref[...]
    b2d = b2d_ref[...]

    def tile8(a):
        return jnp.concatenate([a] * 8, axis=1)

    def pack_r(m):
        return jnp.concatenate(
            [m[ro * _NG:(ro + 1) * _NG, :] for ro in range(8)], axis=1)[None]

    def unpack_r(acc):
        return jnp.concatenate(
            [acc[:, ro * 16:(ro + 1) * 16] for ro in range(8)], axis=0)

    ri = jax.lax.broadcasted_iota(jnp.int32, (128, 128), 0) // 16
    ci = jax.lax.broadcasted_iota(jnp.int32, (128, 128), 1) // 16
    blkmask = ri == ci

    def blockdiag(m):
        rows = jnp.concatenate([m] * 8, axis=0)
        full = jnp.concatenate([rows] * 8, axis=1)
        return jnp.where(blkmask, full, jnp.zeros_like(full))

    li = jax.lax.broadcasted_iota(jnp.int32, (128, 16), 0) % 16
    fi = jax.lax.broadcasted_iota(jnp.int32, (128, 16), 1)
    sel = (li == fi).astype(f32)

    def tile_s(ref, i):
        return ref[pl.ds(i * _S, _S), :]

    relu = lambda t: jnp.maximum(t, 0.0)

    a0 = _dot(x, w1ee[:16, :]) + b1ee
    b0 = _dot(x, w1ee[16:, :])
    b0p = pack_r(b0)
    a0t_ref[...] = tile8(a0)

    def sweepA(i, acc):
        h0 = relu(tile_s(a0t_ref, i)[:, None, :] + b0p)
        return acc + jnp.sum(h0, axis=0)

    accA = jax.lax.fori_loop(0, _NT, sweepA, jnp.zeros((_NG, 128), f32))
    sum_h0 = _dot(jnp.sum(accA, axis=0, keepdims=True), sel)
    m_e0 = _dot(sum_h0 / float(_N * _N), w2ee) + b2ee
    u1h = relu(_dot(m_e0, w1eu[:16, :]) + _dot(u0, w1eu[16:, :]) + b1eu)
    u1 = _dot(u1h, w2eu) + b2eu

    m01 = _dot(w2ee, w1c0e[:16, :])
    wb01 = blockdiag(m01)
    c1 = _dot(b2ee, w1c0e[:16, :]) + _dot(u1, w1c0e[48:, :]) + b1c0e
    a1 = _dot(x, w1c0e[16:32, :]) + c1
    b1_ = _dot(x, w1c0e[32:48, :])
    b1p = pack_r(b1_)
    a1t_ref[...] = tile8(a1)

    def sweepB(i, acc):
        h0 = relu(tile_s(a0t_ref, i)[:, None, :] + b0p)
        t = _edot(h0.reshape(_S * _NG, 128), wb01).reshape(_S, _NG, 128)
        h1 = relu(t + tile_s(a1t_ref, i)[:, None, :] + b1p)
        return acc + jnp.sum(h1, axis=0)

    accB = jax.lax.fori_loop(0, _NT, sweepB, jnp.zeros((_NG, 128), f32))
    hsum1 = unpack_r(accB)
    agg_ev1 = _dot(hsum1 / float(_N), w2c0e) + b2c0e
    m_e1 = _dot(_dot(jnp.sum(accB, axis=0, keepdims=True), sel)
                / float(_N * _N), w2c0e) + b2c0e
    v1h = relu(_dot(x, w1c0v[:16, :]) + _dot(agg_ev1, w1c0v[16:32, :])
               + _dot(u1, w1c0v[32:, :]) + b1c0v)
    v1 = _dot(v1h, w2c0v) + b2c0v
    m_v1 = jnp.sum(v1, axis=0, keepdims=True) / float(_N)
    u2h = relu(_dot(m_e1, w1c0u[:16, :]) + _dot(m_v1, w1c0u[16:32, :])
               + _dot(u1, w1c0u[32:, :]) + b1c0u)
    u2 = _dot(u2h, w2c0u) + b2c0u

    m12 = _dot(w2c0e, w1c1e[:16, :])
    wb12 = blockdiag(m12)
    c2 = _dot(b2c0e, w1c1e[:16, :]) + _dot(u2, w1c1e[48:, :]) + b1c1e
    a2 = _dot(v1, w1c1e[16:32, :]) + c2
    b2_ = _dot(v1, w1c1e[32:48, :])
    b2p = pack_r(b2_)
    a2t_ref[...] = tile8(a2)

    def sweepC(i, acc):
        h0 = relu(tile_s(a0t_ref, i)[:, None, :] + b0p)
        t = _edot(h0.reshape(_S * _NG, 128), wb01).reshape(_S, _NG, 128)
        h1 = relu(t + tile_s(a1t_ref, i)[:, None, :] + b1p)
        t2 = _edot(h1.reshape(_S * _NG, 128), wb12).reshape(_S, _NG, 128)
        h2 = relu(t2 + tile_s(a2t_ref, i)[:, None, :] + b2p)
        return acc + jnp.sum(h2, axis=0)

    accC = jax.lax.fori_loop(0, _NT, sweepC, jnp.zeros((_NG, 128), f32))
    hsum2 = unpack_r(accC)
    agg_ev2 = _dot(hsum2 / float(_N), w2c1e) + b2c1e
    m_e2 = _dot(_dot(jnp.sum(accC, axis=0, keepdims=True), sel)
                / float(_N * _N), w2c1e) + b2c1e
    v2h = relu(_dot(v1, w1c1v[:16, :]) + _dot(agg_ev2, w1c1v[16:32, :])
               + _dot(u2, w1c1v[32:, :]) + b1c1v)
    v2 = _dot(v2h, w2c1v) + b2c1v
    m_v2 = jnp.sum(v2, axis=0, keepdims=True) / float(_N)
    u3h = relu(_dot(m_e2, w1c1u[:16, :]) + _dot(m_v2, w1c1u[16:32, :])
               + _dot(u2, w1c1u[32:, :]) + b1c1u)
    u3 = _dot(u3h, w2c1u) + b2c1u

    outh = relu(_dot(u3, w1d) + b1d)
    out_ref[...] = _dot(outh, w2d) + b2d


def _flatten(x, u, params):
    def mlp(p):
        return (p['W1'], p['b1'].reshape(1, -1), p['W2'], p['b2'].reshape(1, -1))

    c0, c1 = params['cores']
    return ((x, u.reshape(1, -1))
            + mlp(params['enc_e']) + mlp(params['enc_u'])
            + mlp(c0['e']) + mlp(c0['v']) + mlp(c0['u'])
            + mlp(c1['e']) + mlp(c1['v']) + mlp(c1['u'])
            + mlp(params['dec_u']))


def kernel(x, u, params):
    out = pl.pallas_call(
        _gn_body,
        out_shape=jax.ShapeDtypeStruct((1, 8), jnp.float32),
        scratch_shapes=[pltpu.VMEM((_N, 128), jnp.float32)] * 3,
    )(*_flatten(x, u, params))
    return out.reshape(8)


# speedup vs baseline: 1.0632x; 1.0632x over previous
import jax
import jax.numpy as jnp
from jax.experimental import pallas as pl
from jax.experimental.pallas import tpu as pltpu

_N = 1024
_D = 16
_NG = _N // 8
_S = 256
_NT = _N // _S


def _dot(a, b):
    return jax.lax.dot_general(
        a, b, (((a.ndim - 1,), (0,)), ((), ())),
        preferred_element_type=jnp.float32,
        precision=jax.lax.Precision.HIGHEST)


def _edot(a, b):
    return jax.lax.dot_general(
        a, b, (((a.ndim - 1,), (0,)), ((), ())),
        preferred_element_type=jnp.float32)


def _gn_body(x_ref, u_ref,
             w1ee_ref, b1ee_ref, w2ee_ref, b2ee_ref,
             w1eu_ref, b1eu_ref, w2eu_ref, b2eu_ref,
             w1c0e_ref, b1c0e_ref, w2c0e_ref, b2c0e_ref,
             w1c0v_ref, b1c0v_ref, w2c0v_ref, b2c0v_ref,
             w1c0u_ref, b1c0u_ref, w2c0u_ref, b2c0u_ref,
             w1c1e_ref, b1c1e_ref, w2c1e_ref, b2c1e_ref,
             w1c1v_ref, b1c1v_ref, w2c1v_ref, b2c1v_ref,
             w1c1u_ref, b1c1u_ref, w2c1u_ref, b2c1u_ref,
             w1d_ref, b1d_ref, w2d_ref, b2d_ref,
             out_ref, a0t_ref, a1t_ref, a2t_ref):
    f32 = jnp.float32
    x = x_ref[...]
    u0 = u_ref[...]

    w1ee = w1ee_ref[...]
    b1ee = b1ee_ref[...]
    w2ee = w2ee_ref[...]
    b2ee = b2ee_ref[...]
    w1eu = w1eu_ref[...]
    b1eu = b1eu_ref[...]
    w2eu = w2eu_ref[...]
    b2eu = b2eu_ref[...]
    w1c0e = w1c0e_ref[...]
    b1c0e = b1c0e_ref[...]
    w2c0e = w2c0e_ref[...]
    b2c0e = b2c0e_ref[...]
    w1c0v = w1c0v_ref[...]
    b1c0v = b1c0v_ref[...]
    w2c0v = w2c0v_ref[...]
    b2c0v = b2c0v_ref[...]
    w1c0u = w1c0u_ref[...]
    b1c0u = b1c0u_ref[...]
    w2c0u = w2c0u_ref[...]
    b2c0u = b2c0u_ref[...]
    w1c1e = w1c1e_ref[...]
    b1c1e = b1c1e_ref[...]
    w2c1e = w2c1e_ref[...]
    b2c1e = b2c1e_ref[...]
    w1c1v = w1c1v_ref[...]
    b1c1v = b1c1v_ref[...]
    w2c1v = w2c1v_ref[...]
    b2c1v = b2c1v_ref[...]
    w1c1u = w1c1u_ref[...]
    b1c1u = b1c1u_ref[...]
    w2c1u = w2c1u_ref[...]
    b2c1u = b2c1u_ref[...]
    w1d = w1d_ref[...]
    b1d = b1d_ref[...]
    w2d = w2d_ref[...]
    b2d = b2d_ref[...]

    def tile8(a):
        return jnp.concatenate([a] * 8, axis=1)

    def pack_r(m):
        return jnp.concatenate(
            [m[ro * _NG:(ro + 1) * _NG, :] for ro in range(8)], axis=1)[None]

    def unpack_r(acc):
        return jnp.concatenate(
            [acc[:, ro * 16:(ro + 1) * 16] for ro in range(8)], axis=0)

    ri = jax.lax.broadcasted_iota(jnp.int32, (128, 128), 0) // 16
    ci = jax.lax.broadcasted_iota(jnp.int32, (128, 128), 1) // 16
    blkmask = ri == ci

    def blockdiag(m):
        rows = jnp.concatenate([m] * 8, axis=0)
        full = jnp.concatenate([rows] * 8, axis=1)
        return jnp.where(blkmask, full, jnp.zeros_like(full))

    li = jax.lax.broadcasted_iota(jnp.int32, (128, 16), 0) % 16
    fi = jax.lax.broadcasted_iota(jnp.int32, (128, 16), 1)
    sel = (li == fi).astype(f32)

    def tile_s(ref, i):
        return ref[pl.ds(i * _S, _S), :]

    relu = lambda t: jnp.maximum(t, 0.0)

    wx = jnp.concatenate([w1ee[:16, :], w1ee[16:, :], w1c0e[16:32, :],
                          w1c0e[32:48, :], w1c0v[:16, :]], axis=1)
    xg = _dot(x, wx)
    a0 = xg[:, 0:16] + b1ee
    b0p = pack_r(xg[:, 16:32])
    a0t_ref[...] = tile8(a0)

    def sweepA(i, acc):
        h0 = relu(tile_s(a0t_ref, i)[:, None, :] + b0p)
        return acc + jnp.sum(h0, axis=0)

    accA = jax.lax.fori_loop(0, _NT, sweepA, jnp.zeros((_NG, 128), f32))
    sum_h0 = _dot(jnp.sum(accA, axis=0, keepdims=True), sel)
    m_e0 = _dot(sum_h0 / float(_N * _N), w2ee) + b2ee
    u1h = relu(_dot(m_e0, w1eu[:16, :]) + _dot(u0, w1eu[16:, :]) + b1eu)
    u1 = _dot(u1h, w2eu) + b2eu

    m01 = _dot(w2ee, w1c0e[:16, :])
    wb01 = blockdiag(m01)
    c1 = _dot(b2ee, w1c0e[:16, :]) + _dot(u1, w1c0e[48:, :]) + b1c0e
    b1p = pack_r(xg[:, 48:64])
    a1t_ref[...] = tile8(xg[:, 32:48] + c1)

    def sweepB(i, acc):
        h0 = relu(tile_s(a0t_ref, i)[:, None, :] + b0p)
        t = _edot(h0.reshape(_S * _NG, 128), wb01).reshape(_S, _NG, 128)
        h1 = relu(t + tile_s(a1t_ref, i)[:, None, :] + b1p)
        return acc + jnp.sum(h1, axis=0)

    accB = jax.lax.fori_loop(0, _NT, sweepB, jnp.zeros((_NG, 128), f32))
    hsum1 = unpack_r(accB)
    m_e1 = _dot(_dot(jnp.sum(accB, axis=0, keepdims=True), sel)
                / float(_N * _N), w2c0e) + b2c0e
    mh1v = _dot(w2c0e, w1c0v[16:32, :])
    k1v = _dot(b2c0e, w1c0v[16:32, :]) + _dot(u1, w1c0v[32:, :]) + b1c0v
    v1h = relu(xg[:, 64:80] + _dot(hsum1 / float(_N), mh1v) + k1v)
    v1 = _dot(v1h, w2c0v) + b2c0v
    m_v1 = jnp.sum(v1, axis=0, keepdims=True) / float(_N)
    u2h = relu(_dot(m_e1, w1c0u[:16, :]) + _dot(m_v1, w1c0u[16:32, :])
               + _dot(u1, w1c0u[32:, :]) + b1c0u)
    u2 = _dot(u2h, w2c0u) + b2c0u

    m12 = _dot(w2c0e, w1c1e[:16, :])
    wb12 = blockdiag(m12)
    c2 = _dot(b2c0e, w1c1e[:16, :]) + _dot(u2, w1c1e[48:, :]) + b1c1e
    wv = jnp.concatenate([w1c1e[16:32, :], w1c1e[32:48, :],
                          w1c1v[:16, :]], axis=1)
    vg = _dot(v1, wv)
    b2p = pack_r(vg[:, 16:32])
    a2t_ref[...] = tile8(vg[:, 0:16] + c2)

    def sweepC(i, acc):
        h0 = relu(tile_s(a0t_ref, i)[:, None, :] + b0p)
        t = _edot(h0.reshape(_S * _NG, 128), wb01).reshape(_S, _NG, 128)
        h1 = relu(t + tile_s(a1t_ref, i)[:, None, :] + b1p)
        t2 = _edot(h1.reshape(_S * _NG, 128), wb12).reshape(_S, _NG, 128)
        h2 = relu(t2 + tile_s(a2t_ref, i)[:, None, :] + b2p)
        return acc + jnp.sum(h2, axis=0)

    accC = jax.lax.fori_loop(0, _NT, sweepC, jnp.zeros((_NG, 128), f32))
    hsum2 = unpack_r(accC)
    m_e2 = _dot(_dot(jnp.sum(accC, axis=0, keepdims=True), sel)
                / float(_N * _N), w2c1e) + b2c1e
    mh2v = _dot(w2c1e, w1c1v[16:32, :])
    k2v = _dot(b2c1e, w1c1v[16:32, :]) + _dot(u2, w1c1v[32:, :]) + b1c1v
    v2h = relu(vg[:, 32:48] + _dot(hsum2 / float(_N), mh2v) + k2v)
    v2 = _dot(v2h, w2c1v) + b2c1v
    m_v2 = jnp.sum(v2, axis=0, keepdims=True) / float(_N)
    u3h = relu(_dot(m_e2, w1c1u[:16, :]) + _dot(m_v2, w1c1u[16:32, :])
               + _dot(u2, w1c1u[32:, :]) + b1c1u)
    u3 = _dot(u3h, w2c1u) + b2c1u

    outh = relu(_dot(u3, w1d) + b1d)
    out_ref[...] = _dot(outh, w2d) + b2d


def _flatten(x, u, params):
    def mlp(p):
        return (p['W1'], p['b1'].reshape(1, -1), p['W2'], p['b2'].reshape(1, -1))

    c0, c1 = params['cores']
    return ((x, u.reshape(1, -1))
            + mlp(params['enc_e']) + mlp(params['enc_u'])
            + mlp(c0['e']) + mlp(c0['v']) + mlp(c0['u'])
            + mlp(c1['e']) + mlp(c1['v']) + mlp(c1['u'])
            + mlp(params['dec_u']))


def kernel(x, u, params):
    out = pl.pallas_call(
        _gn_body,
        out_shape=jax.ShapeDtypeStruct((1, 8), jnp.float32),
        scratch_shapes=[pltpu.VMEM((_N, 128), jnp.float32)] * 3,
    )(*_flatten(x, u, params))
    return out.reshape(8)


# speedup vs baseline: 1.1766x; 1.1067x over previous
import jax
import jax.numpy as jnp
from jax.experimental import pallas as pl
from jax.experimental.pallas import tpu as pltpu

_N = 1024
_D = 16
_NG = _N // 8
_S = 256
_NT = _N // _S


def _dot(a, b):
    return jax.lax.dot_general(
        a, b, (((a.ndim - 1,), (0,)), ((), ())),
        preferred_element_type=jnp.float32,
        precision=jax.lax.Precision.HIGHEST)


def _edot(a, b):
    return jax.lax.dot_general(
        a, b, (((a.ndim - 1,), (0,)), ((), ())),
        preferred_element_type=jnp.float32).astype(jnp.bfloat16)


def _gn_body(x_ref, u_ref,
             w1ee_ref, b1ee_ref, w2ee_ref, b2ee_ref,
             w1eu_ref, b1eu_ref, w2eu_ref, b2eu_ref,
             w1c0e_ref, b1c0e_ref, w2c0e_ref, b2c0e_ref,
             w1c0v_ref, b1c0v_ref, w2c0v_ref, b2c0v_ref,
             w1c0u_ref, b1c0u_ref, w2c0u_ref, b2c0u_ref,
             w1c1e_ref, b1c1e_ref, w2c1e_ref, b2c1e_ref,
             w1c1v_ref, b1c1v_ref, w2c1v_ref, b2c1v_ref,
             w1c1u_ref, b1c1u_ref, w2c1u_ref, b2c1u_ref,
             w1d_ref, b1d_ref, w2d_ref, b2d_ref,
             out_ref, a0t_ref, a1t_ref, a2t_ref):
    f32 = jnp.float32
    x = x_ref[...]
    u0 = u_ref[...]

    w1ee = w1ee_ref[...]
    b1ee = b1ee_ref[...]
    w2ee = w2ee_ref[...]
    b2ee = b2ee_ref[...]
    w1eu = w1eu_ref[...]
    b1eu = b1eu_ref[...]
    w2eu = w2eu_ref[...]
    b2eu = b2eu_ref[...]
    w1c0e = w1c0e_ref[...]
    b1c0e = b1c0e_ref[...]
    w2c0e = w2c0e_ref[...]
    b2c0e = b2c0e_ref[...]
    w1c0v = w1c0v_ref[...]
    b1c0v = b1c0v_ref[...]
    w2c0v = w2c0v_ref[...]
    b2c0v = b2c0v_ref[...]
    w1c0u = w1c0u_ref[...]
    b1c0u = b1c0u_ref[...]
    w2c0u = w2c0u_ref[...]
    b2c0u = b2c0u_ref[...]
    w1c1e = w1c1e_ref[...]
    b1c1e = b1c1e_ref[...]
    w2c1e = w2c1e_ref[...]
    b2c1e = b2c1e_ref[...]
    w1c1v = w1c1v_ref[...]
    b1c1v = b1c1v_ref[...]
    w2c1v = w2c1v_ref[...]
    b2c1v = b2c1v_ref[...]
    w1c1u = w1c1u_ref[...]
    b1c1u = b1c1u_ref[...]
    w2c1u = w2c1u_ref[...]
    b2c1u = b2c1u_ref[...]
    w1d = w1d_ref[...]
    b1d = b1d_ref[...]
    w2d = w2d_ref[...]
    b2d = b2d_ref[...]

    def tile8(a):
        return jnp.concatenate([a] * 8, axis=1)

    def pack_r(m):
        return jnp.concatenate(
            [m[ro * _NG:(ro + 1) * _NG, :] for ro in range(8)], axis=1)[None]

    def unpack_r(acc):
        return jnp.concatenate(
            [acc[:, ro * 16:(ro + 1) * 16] for ro in range(8)], axis=0)

    ri = jax.lax.broadcasted_iota(jnp.int32, (128, 128), 0) // 16
    ci = jax.lax.broadcasted_iota(jnp.int32, (128, 128), 1) // 16
    blkmask = ri == ci

    def blockdiag(m):
        rows = jnp.concatenate([m] * 8, axis=0)
        full = jnp.concatenate([rows] * 8, axis=1)
        return jnp.where(blkmask, full, jnp.zeros_like(full))

    li = jax.lax.broadcasted_iota(jnp.int32, (128, 16), 0) % 16
    fi = jax.lax.broadcasted_iota(jnp.int32, (128, 16), 1)
    sel = (li == fi).astype(f32)

    def tile_s(ref, i):
        return ref[pl.ds(i * _S, _S), :]

    relu = lambda t: jnp.maximum(t, 0.0)

    wx = jnp.concatenate([w1ee[:16, :], w1ee[16:, :], w1c0e[16:32, :],
                          w1c0e[32:48, :], w1c0v[:16, :]], axis=1)
    xg = _dot(x, wx)
    bf16 = jnp.bfloat16
    a0 = xg[:, 0:16] + b1ee
    b0p = pack_r(xg[:, 16:32]).astype(bf16)
    a0t_ref[...] = tile8(a0).astype(bf16)

    def sweepA(i, acc):
        h0 = relu(tile_s(a0t_ref, i)[:, None, :] + b0p)
        return acc + jnp.sum(h0, axis=0, dtype=f32)

    accA = jax.lax.fori_loop(0, _NT, sweepA, jnp.zeros((_NG, 128), f32))
    sum_h0 = _dot(jnp.sum(accA, axis=0, keepdims=True), sel)
    m_e0 = _dot(sum_h0 / float(_N * _N), w2ee) + b2ee
    u1h = relu(_dot(m_e0, w1eu[:16, :]) + _dot(u0, w1eu[16:, :]) + b1eu)
    u1 = _dot(u1h, w2eu) + b2eu

    m01 = _dot(w2ee, w1c0e[:16, :])
    wb01 = blockdiag(m01)
    c1 = _dot(b2ee, w1c0e[:16, :]) + _dot(u1, w1c0e[48:, :]) + b1c0e
    b1p = pack_r(xg[:, 48:64]).astype(bf16)
    a1t_ref[...] = tile8(xg[:, 32:48] + c1).astype(bf16)

    wb01h = wb01.astype(bf16)

    def sweepB(i, acc):
        h0 = relu(tile_s(a0t_ref, i)[:, None, :] + b0p)
        t = _edot(h0.reshape(_S * _NG, 128), wb01h).reshape(_S, _NG, 128)
        h1 = relu(t + tile_s(a1t_ref, i)[:, None, :] + b1p)
        return acc + jnp.sum(h1, axis=0, dtype=f32)

    accB = jax.lax.fori_loop(0, _NT, sweepB, jnp.zeros((_NG, 128), f32))
    hsum1 = unpack_r(accB)
    m_e1 = _dot(_dot(jnp.sum(accB, axis=0, keepdims=True), sel)
                / float(_N * _N), w2c0e) + b2c0e
    mh1v = _dot(w2c0e, w1c0v[16:32, :])
    k1v = _dot(b2c0e, w1c0v[16:32, :]) + _dot(u1, w1c0v[32:, :]) + b1c0v
    v1h = relu(xg[:, 64:80] + _dot(hsum1 / float(_N), mh1v) + k1v)
    v1 = _dot(v1h, w2c0v) + b2c0v
    m_v1 = jnp.sum(v1, axis=0, keepdims=True) / float(_N)
    u2h = relu(_dot(m_e1, w1c0u[:16, :]) + _dot(m_v1, w1c0u[16:32, :])
               + _dot(u1, w1c0u[32:, :]) + b1c0u)
    u2 = _dot(u2h, w2c0u) + b2c0u

    m12 = _dot(w2c0e, w1c1e[:16, :])
    wb12 = blockdiag(m12)
    c2 = _dot(b2c0e, w1c1e[:16, :]) + _dot(u2, w1c1e[48:, :]) + b1c1e
    wv = jnp.concatenate([w1c1e[16:32, :], w1c1e[32:48, :],
                          w1c1v[:16, :]], axis=1)
    vg = _dot(v1, wv)
    b2p = pack_r(vg[:, 16:32]).astype(bf16)
    a2t_ref[...] = tile8(vg[:, 0:16] + c2).astype(bf16)

    wb12h = wb12.astype(bf16)

    def sweepC(i, acc):
        h0 = relu(tile_s(a0t_ref, i)[:, None, :] + b0p)
        t = _edot(h0.reshape(_S * _NG, 128), wb01h).reshape(_S, _NG, 128)
        h1 = relu(t + tile_s(a1t_ref, i)[:, None, :] + b1p)
        t2 = _edot(h1.reshape(_S * _NG, 128), wb12h).reshape(_S, _NG, 128)
        h2 = relu(t2 + tile_s(a2t_ref, i)[:, None, :] + b2p)
        return acc + jnp.sum(h2, axis=0, dtype=f32)

    accC = jax.lax.fori_loop(0, _NT, sweepC, jnp.zeros((_NG, 128), f32))
    hsum2 = unpack_r(accC)
    m_e2 = _dot(_dot(jnp.sum(accC, axis=0, keepdims=True), sel)
                / float(_N * _N), w2c1e) + b2c1e
    mh2v = _dot(w2c1e, w1c1v[16:32, :])
    k2v = _dot(b2c1e, w1c1v[16:32, :]) + _dot(u2, w1c1v[32:, :]) + b1c1v
    v2h = relu(vg[:, 32:48] + _dot(hsum2 / float(_N), mh2v) + k2v)
    v2 = _dot(v2h, w2c1v) + b2c1v
    m_v2 = jnp.sum(v2, axis=0, keepdims=True) / float(_N)
    u3h = relu(_dot(m_e2, w1c1u[:16, :]) + _dot(m_v2, w1c1u[16:32, :])
               + _dot(u2, w1c1u[32:, :]) + b1c1u)
    u3 = _dot(u3h, w2c1u) + b2c1u

    outh = relu(_dot(u3, w1d) + b1d)
    out_ref[...] = _dot(outh, w2d) + b2d


def _flatten(x, u, params):
    def mlp(p):
        return (p['W1'], p['b1'].reshape(1, -1), p['W2'], p['b2'].reshape(1, -1))

    c0, c1 = params['cores']
    return ((x, u.reshape(1, -1))
            + mlp(params['enc_e']) + mlp(params['enc_u'])
            + mlp(c0['e']) + mlp(c0['v']) + mlp(c0['u'])
            + mlp(c1['e']) + mlp(c1['v']) + mlp(c1['u'])
            + mlp(params['dec_u']))


def kernel(x, u, params):
    out = pl.pallas_call(
        _gn_body,
        out_shape=jax.ShapeDtypeStruct((1, 8), jnp.float32),
        scratch_shapes=[pltpu.VMEM((_N, 128), jnp.bfloat16)] * 3,
    )(*_flatten(x, u, params))
    return out.reshape(8)


# speedup vs baseline: 1.1902x; 1.0116x over previous
import jax
import jax.numpy as jnp
from jax.experimental import pallas as pl
from jax.experimental.pallas import tpu as pltpu

_N = 1024
_D = 16
_NG = _N // 8
_S = 512
_NT = _N // _S


def _dot(a, b):
    return jax.lax.dot_general(
        a, b, (((a.ndim - 1,), (0,)), ((), ())),
        preferred_element_type=jnp.float32,
        precision=jax.lax.Precision.HIGHEST)


def _edot(a, b):
    return jax.lax.dot_general(
        a, b, (((a.ndim - 1,), (0,)), ((), ())),
        preferred_element_type=jnp.float32).astype(jnp.bfloat16)


def _gn_body(x_ref, u_ref,
             w1ee_ref, b1ee_ref, w2ee_ref, b2ee_ref,
             w1eu_ref, b1eu_ref, w2eu_ref, b2eu_ref,
             w1c0e_ref, b1c0e_ref, w2c0e_ref, b2c0e_ref,
             w1c0v_ref, b1c0v_ref, w2c0v_ref, b2c0v_ref,
             w1c0u_ref, b1c0u_ref, w2c0u_ref, b2c0u_ref,
             w1c1e_ref, b1c1e_ref, w2c1e_ref, b2c1e_ref,
             w1c1v_ref, b1c1v_ref, w2c1v_ref, b2c1v_ref,
             w1c1u_ref, b1c1u_ref, w2c1u_ref, b2c1u_ref,
             w1d_ref, b1d_ref, w2d_ref, b2d_ref,
             out_ref, a0t_ref, a1t_ref, a2t_ref):
    f32 = jnp.float32
    x = x_ref[...]
    u0 = u_ref[...]

    w1ee = w1ee_ref[...]
    b1ee = b1ee_ref[...]
    w2ee = w2ee_ref[...]
    b2ee = b2ee_ref[...]
    w1eu = w1eu_ref[...]
    b1eu = b1eu_ref[...]
    w2eu = w2eu_ref[...]
    b2eu = b2eu_ref[...]
    w1c0e = w1c0e_ref[...]
    b1c0e = b1c0e_ref[...]
    w2c0e = w2c0e_ref[...]
    b2c0e = b2c0e_ref[...]
    w1c0v = w1c0v_ref[...]
    b1c0v = b1c0v_ref[...]
    w2c0v = w2c0v_ref[...]
    b2c0v = b2c0v_ref[...]
    w1c0u = w1c0u_ref[...]
    b1c0u = b1c0u_ref[...]
    w2c0u = w2c0u_ref[...]
    b2c0u = b2c0u_ref[...]
    w1c1e = w1c1e_ref[...]
    b1c1e = b1c1e_ref[...]
    w2c1e = w2c1e_ref[...]
    b2c1e = b2c1e_ref[...]
    w1c1v = w1c1v_ref[...]
    b1c1v = b1c1v_ref[...]
    w2c1v = w2c1v_ref[...]
    b2c1v = b2c1v_ref[...]
    w1c1u = w1c1u_ref[...]
    b1c1u = b1c1u_ref[...]
    w2c1u = w2c1u_ref[...]
    b2c1u = b2c1u_ref[...]
    w1d = w1d_ref[...]
    b1d = b1d_ref[...]
    w2d = w2d_ref[...]
    b2d = b2d_ref[...]

    def tile8(a):
        return jnp.concatenate([a] * 8, axis=1)

    def pack_r(m):
        return jnp.concatenate(
            [m[ro * _NG:(ro + 1) * _NG, :] for ro in range(8)], axis=1)[None]

    def unpack_r(acc):
        return jnp.concatenate(
            [acc[:, ro * 16:(ro + 1) * 16] for ro in range(8)], axis=0)

    ri = jax.lax.broadcasted_iota(jnp.int32, (128, 128), 0) // 16
    ci = jax.lax.broadcasted_iota(jnp.int32, (128, 128), 1) // 16
    blkmask = ri == ci

    def blockdiag(m):
        rows = jnp.concatenate([m] * 8, axis=0)
        full = jnp.concatenate([rows] * 8, axis=1)
        return jnp.where(blkmask, full, jnp.zeros_like(full))

    li = jax.lax.broadcasted_iota(jnp.int32, (128, 16), 0) % 16
    fi = jax.lax.broadcasted_iota(jnp.int32, (128, 16), 1)
    sel = (li == fi).astype(f32)

    def tile_s(ref, i):
        return ref[pl.ds(i * _S, _S), :]

    relu = lambda t: jnp.maximum(t, 0.0)

    wx = jnp.concatenate([w1ee[:16, :], w1ee[16:, :], w1c0e[16:32, :],
                          w1c0e[32:48, :], w1c0v[:16, :]], axis=1)
    xg = _dot(x, wx)
    bf16 = jnp.bfloat16
    a0 = xg[:, 0:16] + b1ee
    b0p = pack_r(xg[:, 16:32]).astype(bf16)
    a0t_ref[...] = tile8(a0).astype(bf16)

    def sweepA(i, acc):
        h0 = relu(tile_s(a0t_ref, i)[:, None, :] + b0p)
        return acc + jnp.sum(h0, axis=0, dtype=f32)

    accA = jax.lax.fori_loop(0, _NT, sweepA, jnp.zeros((_NG, 128), f32))
    sum_h0 = _dot(jnp.sum(accA, axis=0, keepdims=True), sel)
    m_e0 = _dot(sum_h0 / float(_N * _N), w2ee) + b2ee
    u1h = relu(_dot(m_e0, w1eu[:16, :]) + _dot(u0, w1eu[16:, :]) + b1eu)
    u1 = _dot(u1h, w2eu) + b2eu

    m01 = _dot(w2ee, w1c0e[:16, :])
    wb01 = blockdiag(m01)
    c1 = _dot(b2ee, w1c0e[:16, :]) + _dot(u1, w1c0e[48:, :]) + b1c0e
    b1p = pack_r(xg[:, 48:64]).astype(bf16)
    a1t_ref[...] = tile8(xg[:, 32:48] + c1).astype(bf16)

    wb01h = wb01.astype(bf16)

    def sweepB(i, acc):
        h0 = relu(tile_s(a0t_ref, i)[:, None, :] + b0p)
        t = _edot(h0.reshape(_S * _NG, 128), wb01h).reshape(_S, _NG, 128)
        h1 = relu(t + tile_s(a1t_ref, i)[:, None, :] + b1p)
        return acc + jnp.sum(h1, axis=0, dtype=f32)

    accB = jax.lax.fori_loop(0, _NT, sweepB, jnp.zeros((_NG, 128), f32))
    hsum1 = unpack_r(accB)
    m_e1 = _dot(_dot(jnp.sum(accB, axis=0, keepdims=True), sel)
                / float(_N * _N), w2c0e) + b2c0e
    mh1v = _dot(w2c0e, w1c0v[16:32, :])
    k1v = _dot(b2c0e, w1c0v[16:32, :]) + _dot(u1, w1c0v[32:, :]) + b1c0v
    v1h = relu(xg[:, 64:80] + _dot(hsum1 / float(_N), mh1v) + k1v)
    v1 = _dot(v1h, w2c0v) + b2c0v
    m_v1 = jnp.sum(v1, axis=0, keepdims=True) / float(_N)
    u2h = relu(_dot(m_e1, w1c0u[:16, :]) + _dot(m_v1, w1c0u[16:32, :])
               + _dot(u1, w1c0u[32:, :]) + b1c0u)
    u2 = _dot(u2h, w2c0u) + b2c0u

    m12 = _dot(w2c0e, w1c1e[:16, :])
    wb12 = blockdiag(m12)
    c2 = _dot(b2c0e, w1c1e[:16, :]) + _dot(u2, w1c1e[48:, :]) + b1c1e
    wv = jnp.concatenate([w1c1e[16:32, :], w1c1e[32:48, :],
                          w1c1v[:16, :]], axis=1)
    vg = _dot(v1, wv)
    b2p = pack_r(vg[:, 16:32]).astype(bf16)
    a2t_ref[...] = tile8(vg[:, 0:16] + c2).astype(bf16)

    wb12h = wb12.astype(bf16)

    def sweepC(i, acc):
        h0 = relu(tile_s(a0t_ref, i)[:, None, :] + b0p)
        t = _edot(h0.reshape(_S * _NG, 128), wb01h).reshape(_S, _NG, 128)
        h1 = relu(t + tile_s(a1t_ref, i)[:, None, :] + b1p)
        t2 = _edot(h1.reshape(_S * _NG, 128), wb12h).reshape(_S, _NG, 128)
        h2 = relu(t2 + tile_s(a2t_ref, i)[:, None, :] + b2p)
        return acc + jnp.sum(h2, axis=0, dtype=f32)

    accC = jax.lax.fori_loop(0, _NT, sweepC, jnp.zeros((_NG, 128), f32))
    hsum2 = unpack_r(accC)
    m_e2 = _dot(_dot(jnp.sum(accC, axis=0, keepdims=True), sel)
                / float(_N * _N), w2c1e) + b2c1e
    mh2v = _dot(w2c1e, w1c1v[16:32, :])
    k2v = _dot(b2c1e, w1c1v[16:32, :]) + _dot(u2, w1c1v[32:, :]) + b1c1v
    v2h = relu(vg[:, 32:48] + _dot(hsum2 / float(_N), mh2v) + k2v)
    v2 = _dot(v2h, w2c1v) + b2c1v
    m_v2 = jnp.sum(v2, axis=0, keepdims=True) / float(_N)
    u3h = relu(_dot(m_e2, w1c1u[:16, :]) + _dot(m_v2, w1c1u[16:32, :])
               + _dot(u2, w1c1u[32:, :]) + b1c1u)
    u3 = _dot(u3h, w2c1u) + b2c1u

    outh = relu(_dot(u3, w1d) + b1d)
    out_ref[...] = _dot(outh, w2d) + b2d


def _flatten(x, u, params):
    def mlp(p):
        return (p['W1'], p['b1'].reshape(1, -1), p['W2'], p['b2'].reshape(1, -1))

    c0, c1 = params['cores']
    return ((x, u.reshape(1, -1))
            + mlp(params['enc_e']) + mlp(params['enc_u'])
            + mlp(c0['e']) + mlp(c0['v']) + mlp(c0['u'])
            + mlp(c1['e']) + mlp(c1['v']) + mlp(c1['u'])
            + mlp(params['dec_u']))


def kernel(x, u, params):
    out = pl.pallas_call(
        _gn_body,
        out_shape=jax.ShapeDtypeStruct((1, 8), jnp.float32),
        scratch_shapes=[pltpu.VMEM((_N, 128), jnp.bfloat16)] * 3,
    )(*_flatten(x, u, params))
    return out.reshape(8)
